# baseline (speedup 1.0000x reference)
import jax
import jax.numpy as jnp
from jax import lax
from jax.experimental import pallas as pl
from jax.experimental.pallas import tpu as pltpu
from jax.experimental.pallas import tpu_sc as plsc

_V = 1000000
_D = 16
_L = 200
_LC = 50
_FE = 6
_T1 = 16


def _sc_gather(table, w1v, idx, n_chunks, k):
    N = idx.shape[0]
    info = plsc.get_sparse_core_info()
    NC = info.num_cores
    NW = NC * info.num_subcores
    n = N // NW
    CR = k * 128
    assert n == n_chunks * CR, (N, NW, n, n_chunks, k)
    idx2 = idx.reshape(NW, n // 128, 128)
    mesh = plsc.VectorSubcoreMesh(core_axis_name="c", subcore_axis_name="s")
    with_w1 = w1v is not None

    out_types = [jax.ShapeDtypeStruct((N, _D), jnp.float32)]
    if with_w1:
        out_types.append(jax.ShapeDtypeStruct((N,), jnp.float32))
    scratch = [pltpu.VMEM((n // 128, 128), jnp.int32),
               pltpu.VMEM((CR, _D), jnp.float32)]
    if with_w1:
        scratch.append(pltpu.VMEM((CR,), jnp.float32))
    scratch.append(pltpu.SemaphoreType.DMA)
    scratch.append(pltpu.SemaphoreType.DMA)

    def body(*refs):
        if with_w1:
            (table_h, w1_h, idx_h, out_h, w1o_h,
             idx_v, rows_v, w1_v, sem, sem2) = refs
        else:
            (table_h, idx_h, out_h, idx_v, rows_v, sem, sem2) = refs
            w1_h = w1o_h = w1_v = None
        wid = lax.axis_index("s") * NC + lax.axis_index("c")
        pltpu.sync_copy(idx_h.at[wid], idx_v)
        for c in range(n_chunks):
            cps = []
            for i in range(k):
                cps.append(pltpu.async_copy(
                    table_h.at[idx_v.at[c * k + i]],
                    rows_v.at[pl.ds(i * 128, 128)], sem))
                if with_w1:
                    cps.append(pltpu.async_copy(
                        w1_h.at[idx_v.at[c * k + i]],
                        w1_v.at[pl.ds(i * 128, 128)], sem2))
            for cp in cps:
                cp.wait()
            base = wid * n + c * CR
            pltpu.sync_copy(rows_v, out_h.at[pl.ds(base, CR)])
            if with_w1:
                pltpu.sync_copy(w1_v, w1o_h.at[pl.ds(base, CR)])

    fn = pl.kernel(body,
                   out_type=tuple(out_types) if with_w1 else out_types[0],
                   mesh=mesh, scratch_types=scratch,
                   compiler_params=pltpu.CompilerParams(use_tc_tiling_on_sc=False))
    if with_w1:
        return fn(table, w1v, idx2)
    return fn(table, idx2)


def _topk_body(e_ref, xs_ref, sl_ref, out_ref):
    f32 = jnp.float32
    i32 = jnp.int32
    hi = lax.Precision.HIGHEST
    E = e_ref[...]
    R = E.shape[0]
    BB = R * 128 // (_L * _D)
    RP = R // BB
    sl = sl_ref[...]
    slB = jnp.broadcast_to(sl[:, None, :], (BB, RP, 1)).reshape(R, 1)
    r_i = lax.broadcasted_iota(i32, (R, 128), 0)
    c_i = lax.broadcasted_iota(i32, (R, 128), 1)
    lidx = (r_i % RP) * 8 + c_i // _D
    masked = jnp.where(lidx == slB, E, 0.0)
    c128 = ((lax.broadcasted_iota(i32, (128, _D), 0) % _D)
            == lax.broadcasted_iota(i32, (128, _D), 1)).astype(f32)
    esrow = jnp.dot(masked, c128, preferred_element_type=f32, precision=hi)
    es16 = jnp.sum(esrow.reshape(BB, RP, _D), axis=1)
    t16 = ((lax.broadcasted_iota(i32, (_D, 128), 1) % _D)
           == lax.broadcasted_iota(i32, (_D, 128), 0)).astype(f32)
    esT = jnp.dot(es16, t16, preferred_element_type=f32, precision=hi)
    esB = jnp.broadcast_to(esT[:, None, :], (BB, RP, 128)).reshape(R, 128)
    m8 = ((lax.broadcasted_iota(i32, (128, 8), 0) // _D)
          == lax.broadcasted_iota(i32, (128, 8), 1)).astype(f32)
    dot8 = jnp.dot(E * esB, m8, preferred_element_type=f32, precision=hi)
    ss8 = jnp.dot(E * E, m8, preferred_element_type=f32, precision=hi)
    ssself = jnp.sum(es16 * es16, axis=1, keepdims=True)

    g8 = ((lax.broadcasted_iota(i32, (8, _L), 1) % 8)
          == lax.broadcasted_iota(i32, (8, _L), 0)).astype(f32)
    m25 = (lax.broadcasted_iota(i32, (RP, _L), 0)
           == lax.broadcasted_iota(i32, (RP, _L), 1) // 8).astype(f32)

    def expand(a8):
        aw = jnp.dot(a8, g8, preferred_element_type=f32, precision=hi)
        return jnp.sum(aw.reshape(BB, RP, _L) * m25[None], axis=1)

    dot = expand(dot8)
    ssall = expand(ss8)
    sim = dot / (jnp.sqrt(ssself + 1e-8) * jnp.sqrt(ssall + 1e-8))
    li = lax.broadcasted_iota(i32, (BB, _L), 1)
    s = jnp.where(li < sl, sim, -2.0)
    sel = jnp.zeros((BB, _L), jnp.bool_)
    for _ in range(15):
        m = jnp.max(s, axis=1, keepdims=True)
        first = jnp.min(jnp.where(s == m, li, _L), axis=1, keepdims=True)
        pick = li == first
        sel = jnp.logical_or(sel, pick)
        s = jnp.where(pick, -3.0, s)
    sel = jnp.logical_or(sel, li == sl)
    lt = (lax.broadcasted_iota(i32, (_L, _L), 0)
          <= lax.broadcasted_iota(i32, (_L, _L), 1)).astype(f32)
    rank = jnp.dot(sel.astype(f32), lt, preferred_element_type=f32).astype(i32) - 1
    xs = xs_ref[...]
    for r in range(_T1):
        pick_r = jnp.logical_and(sel, rank == r)
        v = jnp.sum(jnp.where(pick_r[:, None, :], xs, 0.0), axis=2)
        out_ref[:, r, :] = v.astype(i32)


def _main_body(et_ref, w1t_ref, ec_ref, w1c_ref, ids0_ref, xc0_ref,
               ww_ref, wb_ref, m1_ref, b1_ref, m2_ref, b2_ref,
               fw_ref, fb_ref, out_ref):
    f32 = jnp.float32
    Ww = ww_ref[...]
    Wb = wb_ref[...]

    def pnn(e, w1f):
        acc = jnp.dot(w1f, Ww[0:_FE], preferred_element_type=f32)
        acc += jnp.dot(e, Ww[_FE:_FE + _FE * _D], preferred_element_type=f32)
        kidx = _FE + _FE * _D
        ei = [e[:, _D * i:_D * (i + 1)] for i in range(_FE)]
        for i in range(_FE):
            for j in range(i + 1, _FE):
                pij = jnp.sum(ei[i] * ei[j], axis=1, keepdims=True)
                acc += pij * Ww[kidx:kidx + 1, :]
                kidx += 1
        return acc + Wb

    BB = ids0_ref.shape[0]
    hx2 = pnn(et_ref[...], w1t_ref[...])
    hc2 = pnn(ec_ref[...], w1c_ref[...])
    hx3 = hx2.reshape(BB, _T1, 64)
    hc3 = hc2.reshape(BB, _LC, 64)

    def attn(h3, mask):
        q = h3[:, h3.shape[1] - 1, :]
        sc = jnp.sum(h3 * q[:, None, :], axis=2) / 8.0
        sc = jnp.where(mask, sc, -1e9)
        sc = sc - jnp.max(sc, axis=1, keepdims=True)
        a = jnp.exp(sc)
        a = a / jnp.sum(a, axis=1, keepdims=True)
        return jnp.sum(a[:, :, None] * h3, axis=1)

    mask1 = ids0_ref[...] != _V
    mask2 = xc0_ref[...] != _V
    h1 = attn(hx3, mask1)
    h2 = attn(hc3, mask2)
    xself = hx3[:, _T1 - 1, :]
    h = jnp.concatenate([xself, h1, h2], axis=1)
    h = jnp.maximum(jnp.dot(h, m1_ref[...], preferred_element_type=f32)
                    + b1_ref[...], 0.0)
    h = jnp.maximum(jnp.dot(h, m2_ref[...], preferred_element_type=f32)
                    + b2_ref[...], 0.0)
    out_ref[...] = jnp.dot(h, fw_ref[...], preferred_element_type=f32) + fb_ref[...]


def kernel(x, x_continuous, self_loc, y, emb, w1, W_w, W_b,
           mlp_w1, mlp_b1, mlp_w2, mlp_b2, fc_w, fc_b):
    B, L, F = x.shape
    cat = x[:, :, F - 3].reshape(B * L)
    w1r = w1.reshape(-1)
    e_all = _sc_gather(emb, None, cat, n_chunks=2, k=25)
    xc_ids = x_continuous[:, :, :_FE].reshape(B * _LC * _FE)
    ec, w1c = _sc_gather(emb, w1r, xc_ids, n_chunks=3, k=25)

    e2 = e_all.reshape(B * L * _D // 128, 128)
    xs_t = x[:, :, :_FE].astype(jnp.float32).transpose(0, 2, 1)
    sl2 = self_loc.reshape(B, 1)
    BB = 128
    ids = pl.pallas_call(
        _topk_body,
        grid=(B // BB,),
        in_specs=[pl.BlockSpec((BB * L * _D // 128, 128), lambda i: (i, 0)),
                  pl.BlockSpec((BB, _FE, L), lambda i: (i, 0, 0)),
                  pl.BlockSpec((BB, 1), lambda i: (i, 0))],
        out_specs=pl.BlockSpec((BB, _T1, _FE), lambda i: (i, 0, 0)),
        out_shape=jax.ShapeDtypeStruct((B, _T1, _FE), jnp.int32),
    )(e2, xs_t, sl2)

    top_ids = ids.reshape(B * _T1 * _FE)
    et, w1t = _sc_gather(emb, w1r, top_ids, n_chunks=1, k=24)

    et2 = et.reshape(B * _T1, _FE * _D)
    w1t2 = w1t.reshape(B * _T1, _FE)
    ec2 = ec.reshape(B * _LC, _FE * _D)
    w1c2 = w1c.reshape(B * _LC, _FE)
    ids0 = ids[:, :, 0]
    xc0 = x_continuous[:, :, 0]
    wb2 = W_b.reshape(1, -1)
    b12 = mlp_b1.reshape(1, -1)
    b22 = mlp_b2.reshape(1, -1)
    fb2 = fc_b.reshape(1, 1)
    out = pl.pallas_call(
        _main_body,
        grid=(B // BB,),
        in_specs=[pl.BlockSpec((BB * _T1, _FE * _D), lambda i: (i, 0)),
                  pl.BlockSpec((BB * _T1, _FE), lambda i: (i, 0)),
                  pl.BlockSpec((BB * _LC, _FE * _D), lambda i: (i, 0)),
                  pl.BlockSpec((BB * _LC, _FE), lambda i: (i, 0)),
                  pl.BlockSpec((BB, _T1), lambda i: (i, 0)),
                  pl.BlockSpec((BB, _LC), lambda i: (i, 0)),
                  pl.BlockSpec((117, 64), lambda i: (0, 0)),
                  pl.BlockSpec((1, 64), lambda i: (0, 0)),
                  pl.BlockSpec((192, 200), lambda i: (0, 0)),
                  pl.BlockSpec((1, 200), lambda i: (0, 0)),
                  pl.BlockSpec((200, 80), lambda i: (0, 0)),
                  pl.BlockSpec((1, 80), lambda i: (0, 0)),
                  pl.BlockSpec((80, 1), lambda i: (0, 0)),
                  pl.BlockSpec((1, 1), lambda i: (0, 0))],
        out_specs=pl.BlockSpec((BB, 1), lambda i: (i, 0)),
        out_shape=jax.ShapeDtypeStruct((B, 1), jnp.float32),
    )(et2, w1t2, ec2, w1c2, ids0, xc0,
      W_w, wb2, mlp_w1, b12, mlp_w2, b22, fc_w, fb2)
    return out


# speedup vs baseline: 1.1633x; 1.1633x over previous
import jax
import jax.numpy as jnp
from jax import lax
from jax.experimental import pallas as pl
from jax.experimental.pallas import tpu as pltpu
from jax.experimental.pallas import tpu_sc as plsc

_V = 1000000
_D = 16
_L = 200
_LC = 50
_FE = 6
_T1 = 16


def _sc_gather(table, w1v, idx, n_chunks, k):
    N = idx.shape[0]
    info = plsc.get_sparse_core_info()
    NC = info.num_cores
    NW = NC * info.num_subcores
    n = N // NW
    CR = k * 128
    assert n == n_chunks * CR, (N, NW, n, n_chunks, k)
    idx2 = idx.reshape(NW, n // 128, 128)
    mesh = plsc.VectorSubcoreMesh(core_axis_name="c", subcore_axis_name="s")
    with_w1 = w1v is not None

    out_types = [jax.ShapeDtypeStruct((N, _D), jnp.float32)]
    if with_w1:
        out_types.append(jax.ShapeDtypeStruct((N,), jnp.float32))
    scratch = [pltpu.VMEM((n // 128, 128), jnp.int32),
               pltpu.VMEM((CR, _D), jnp.float32)]
    if with_w1:
        scratch.append(pltpu.VMEM((CR,), jnp.float32))
    scratch.append(pltpu.SemaphoreType.DMA)
    scratch.append(pltpu.SemaphoreType.DMA)

    def body(*refs):
        if with_w1:
            (table_h, w1_h, idx_h, out_h, w1o_h,
             idx_v, rows_v, w1_v, sem, sem2) = refs
        else:
            (table_h, idx_h, out_h, idx_v, rows_v, sem, sem2) = refs
            w1_h = w1o_h = w1_v = None
        wid = lax.axis_index("s") * NC + lax.axis_index("c")
        pltpu.sync_copy(idx_h.at[wid], idx_v)
        for c in range(n_chunks):
            cps = []
            for i in range(k):
                cps.append(pltpu.async_copy(
                    table_h.at[idx_v.at[c * k + i]],
                    rows_v.at[pl.ds(i * 128, 128)], sem))
                if with_w1:
                    cps.append(pltpu.async_copy(
                        w1_h.at[idx_v.at[c * k + i]],
                        w1_v.at[pl.ds(i * 128, 128)], sem2))
            for cp in cps:
                cp.wait()
            base = wid * n + c * CR
            pltpu.sync_copy(rows_v, out_h.at[pl.ds(base, CR)])
            if with_w1:
                pltpu.sync_copy(w1_v, w1o_h.at[pl.ds(base, CR)])

    fn = pl.kernel(body,
                   out_type=tuple(out_types) if with_w1 else out_types[0],
                   mesh=mesh, scratch_types=scratch,
                   compiler_params=pltpu.CompilerParams(use_tc_tiling_on_sc=False))
    if with_w1:
        return fn(table, w1v, idx2)
    return fn(table, idx2)


def _topk_body(e_ref, xs_ref, sl_ref, out_ref):
    f32 = jnp.float32
    i32 = jnp.int32
    hi = lax.Precision.HIGHEST
    E = e_ref[...]
    sl = sl_ref[...]
    BB = E.shape[0]
    LD = _L * _D
    lane = lax.broadcasted_iota(i32, (BB, LD), 1)
    selfexp = (lane // _D) == sl
    masked = jnp.where(selfexp, E, 0.0)
    c_m = ((lax.broadcasted_iota(i32, (LD, _D), 0) % _D)
           == lax.broadcasted_iota(i32, (LD, _D), 1)).astype(f32)
    es16 = jnp.dot(masked, c_m, preferred_element_type=f32, precision=hi)
    r_m = ((lax.broadcasted_iota(i32, (_D, LD), 1) % _D)
           == lax.broadcasted_iota(i32, (_D, LD), 0)).astype(f32)
    es_exp = jnp.dot(es16, r_m, preferred_element_type=f32, precision=hi)
    s_m = ((lax.broadcasted_iota(i32, (LD, _L), 0) // _D)
           == lax.broadcasted_iota(i32, (LD, _L), 1)).astype(f32)
    dot = jnp.dot(E * es_exp, s_m, preferred_element_type=f32, precision=hi)
    ssall = jnp.dot(E * E, s_m, preferred_element_type=f32, precision=hi)
    ssself = jnp.sum(es16 * es16, axis=1, keepdims=True)
    sim = dot / (jnp.sqrt(ssself + 1e-8) * jnp.sqrt(ssall + 1e-8))
    li = lax.broadcasted_iota(i32, (BB, _L), 1)
    s = jnp.where(li < sl, sim, -2.0)
    sel = jnp.zeros((BB, _L), jnp.bool_)
    for _ in range(15):
        m = jnp.max(s, axis=1, keepdims=True)
        first = jnp.min(jnp.where(s == m, li, _L), axis=1, keepdims=True)
        pick = li == first
        sel = jnp.logical_or(sel, pick)
        s = jnp.where(pick, -3.0, s)
    sel = jnp.logical_or(sel, li == sl)
    lt = (lax.broadcasted_iota(i32, (_L, _L), 0)
          <= lax.broadcasted_iota(i32, (_L, _L), 1)).astype(f32)
    rank = jnp.dot(sel.astype(f32), lt, preferred_element_type=f32).astype(i32) - 1
    xs = xs_ref[...]
    for r in range(8 * _T1 // 8):
        pick_r = jnp.logical_and(sel, rank == r)
        v = jnp.sum(jnp.where(pick_r[:, None, :], xs, 0.0), axis=2)
        out_ref[:, r, :] = v.astype(i32)


def _main_body(et_ref, w1t_ref, ec_ref, w1c_ref, ids0_ref, xc0_ref,
               ww_ref, wb_ref, m1_ref, b1_ref, m2_ref, b2_ref,
               fw_ref, fb_ref, out_ref):
    f32 = jnp.float32
    Ww = ww_ref[...]
    Wb = wb_ref[...]

    z2 = jnp.zeros((2, 64), f32)
    z32 = jnp.zeros((32, 64), f32)
    wlin = jnp.concatenate([Ww[0:_FE], z2], axis=0)
    wflat = jnp.concatenate([Ww[_FE:_FE + _FE * _D], z32], axis=0)

    def pnn(e, w1f):
        acc = jnp.dot(w1f, wlin, preferred_element_type=f32)
        acc += jnp.dot(e, wflat, preferred_element_type=f32)
        kidx = _FE + _FE * _D
        ei = [e[:, _D * i:_D * (i + 1)] for i in range(_FE)]
        for i in range(_FE):
            for j in range(i + 1, _FE):
                pij = jnp.sum(ei[i] * ei[j], axis=1, keepdims=True)
                acc += pij * Ww[kidx:kidx + 1, :]
                kidx += 1
        return acc + Wb

    BB = ids0_ref.shape[0]
    hx2 = pnn(et_ref[...], w1t_ref[...])
    hc2 = pnn(ec_ref[...], w1c_ref[...])
    hx3 = hx2.reshape(BB, _T1, 64)
    hc3 = hc2.reshape(BB, _LC, 64)

    def attn(h3, mask):
        q = h3[:, h3.shape[1] - 1, :]
        sc = jnp.sum(h3 * q[:, None, :], axis=2) / 8.0
        sc = jnp.where(mask, sc, -1e9)
        sc = sc - jnp.max(sc, axis=1, keepdims=True)
        a = jnp.exp(sc)
        a = a / jnp.sum(a, axis=1, keepdims=True)
        return jnp.sum(a[:, :, None] * h3, axis=1)

    mask1 = ids0_ref[...] != _V
    mask2 = xc0_ref[...] != _V
    h1 = attn(hx3, mask1)
    h2 = attn(hc3, mask2)
    xself = hx3[:, _T1 - 1, :]
    h = jnp.concatenate([xself, h1, h2], axis=1)
    h = jnp.maximum(jnp.dot(h, m1_ref[...], preferred_element_type=f32)
                    + b1_ref[...], 0.0)
    h = jnp.maximum(jnp.dot(h, m2_ref[...], preferred_element_type=f32)
                    + b2_ref[...], 0.0)
    out_ref[...] = jnp.dot(h, fw_ref[...], preferred_element_type=f32) + fb_ref[...]


def kernel(x, x_continuous, self_loc, y, emb, w1, W_w, W_b,
           mlp_w1, mlp_b1, mlp_w2, mlp_b2, fc_w, fc_b):
    B, L, F = x.shape
    cat = x[:, :, F - 3].reshape(B * L)
    w1r = w1.reshape(-1)
    e_all = _sc_gather(emb, None, cat, n_chunks=2, k=25)
    xc_ids = x_continuous.reshape(B * _LC * F)
    ec, w1c = _sc_gather(emb, w1r, xc_ids, n_chunks=4, k=25)

    e2 = e_all.reshape(B, L * _D)
    xs_t = x.astype(jnp.float32).transpose(0, 2, 1)
    sl2 = self_loc.reshape(B, 1)
    BB = 128
    ids = pl.pallas_call(
        _topk_body,
        grid=(B // BB,),
        in_specs=[pl.BlockSpec((BB, L * _D), lambda i: (i, 0)),
                  pl.BlockSpec((BB, F, L), lambda i: (i, 0, 0)),
                  pl.BlockSpec((BB, 1), lambda i: (i, 0))],
        out_specs=pl.BlockSpec((BB, _T1, F), lambda i: (i, 0, 0)),
        out_shape=jax.ShapeDtypeStruct((B, _T1, F), jnp.int32),
    )(e2, xs_t, sl2)

    top_ids = ids.reshape(B * _T1 * F)
    et, w1t = _sc_gather(emb, w1r, top_ids, n_chunks=2, k=16)

    et2 = et.reshape(B * _T1, F * _D)
    w1t2 = w1t.reshape(B * _T1, F)
    ec2 = ec.reshape(B * _LC, F * _D)
    w1c2 = w1c.reshape(B * _LC, F)
    ids0 = ids[:, :, 0]
    xc0 = x_continuous[:, :, 0]
    wb2 = W_b.reshape(1, -1)
    b12 = mlp_b1.reshape(1, -1)
    b22 = mlp_b2.reshape(1, -1)
    fb2 = fc_b.reshape(1, 1)
    out = pl.pallas_call(
        _main_body,
        grid=(B // BB,),
        in_specs=[pl.BlockSpec((BB * _T1, 8 * _D), lambda i: (i, 0)),
                  pl.BlockSpec((BB * _T1, 8), lambda i: (i, 0)),
                  pl.BlockSpec((BB * _LC, 8 * _D), lambda i: (i, 0)),
                  pl.BlockSpec((BB * _LC, 8), lambda i: (i, 0)),
                  pl.BlockSpec((BB, _T1), lambda i: (i, 0)),
                  pl.BlockSpec((BB, _LC), lambda i: (i, 0)),
                  pl.BlockSpec((117, 64), lambda i: (0, 0)),
                  pl.BlockSpec((1, 64), lambda i: (0, 0)),
                  pl.BlockSpec((192, 200), lambda i: (0, 0)),
                  pl.BlockSpec((1, 200), lambda i: (0, 0)),
                  pl.BlockSpec((200, 80), lambda i: (0, 0)),
                  pl.BlockSpec((1, 80), lambda i: (0, 0)),
                  pl.BlockSpec((80, 1), lambda i: (0, 0)),
                  pl.BlockSpec((1, 1), lambda i: (0, 0))],
        out_specs=pl.BlockSpec((BB, 1), lambda i: (i, 0)),
        out_shape=jax.ShapeDtypeStruct((B, 1), jnp.float32),
    )(et2, w1t2, ec2, w1c2, ids0, xc0,
      W_w, wb2, mlp_w1, b12, mlp_w2, b22, fc_w, fb2)
    return out
